# baseline (speedup 1.0000x reference)
import functools

import jax
import jax.numpy as jnp
from jax import lax
from jax.experimental import pallas as pl
from jax.experimental.pallas import tpu as pltpu
from jax.experimental.pallas import tpu_sc as plsc

N_VERTS = 3889
N_CENTER = 889
N_LEFT = 1500
N_SD = 20
N_FIXED = 10
SEC = 32
OFF = 10
ROW = 128
PAD_B = 4096

_info = plsc.get_sparse_core_info()
_NC = _info.num_cores
_NS = _info.num_subcores
_NW = _NC * _NS
_BPW = PAD_B // _NW


def _build_table_body(params_ref, idx_ref, tab_ref, idxp_ref):
    a, b = N_CENTER, N_CENTER + N_LEFT
    c0 = params_ref[0:889, :]
    c2 = params_ref[889:1778, :]
    l0 = params_ref[1778:3278, :]
    l1 = params_ref[3278:4778, :]
    l2 = params_ref[4778:6278, :]
    s0, s1, s2 = OFF, SEC + OFF, 2 * SEC + OFF
    tab_ref[0:a, s0:s0 + N_SD] = c0
    tab_ref[0:a, s1:s1 + N_SD] = jnp.zeros((N_CENTER, N_SD), jnp.float32)
    tab_ref[0:a, s2:s2 + N_SD] = c2
    tab_ref[a:b, s0:s0 + N_SD] = l0
    tab_ref[a:b, s1:s1 + N_SD] = l1
    tab_ref[a:b, s2:s2 + N_SD] = l2
    tab_ref[b:N_VERTS, s0:s0 + N_SD] = l0
    tab_ref[b:N_VERTS, s1:s1 + N_SD] = -l1
    tab_ref[b:N_VERTS, s2:s2 + N_SD] = l2
    ipad = jnp.concatenate(
        [idx_ref[...], jnp.zeros((PAD_B - N_VERTS,), jnp.int32)], axis=0)
    idxp_ref[...] = ipad.reshape(_NW, _BPW)


_sc_mesh = plsc.VectorSubcoreMesh(core_axis_name="c", subcore_axis_name="s")


@functools.partial(
    pl.kernel,
    mesh=_sc_mesh,
    out_type=jax.ShapeDtypeStruct((PAD_B, 3, ROW), jnp.float32),
    scratch_types=[
        pltpu.VMEM((_BPW,), jnp.int32),
        pltpu.VMEM((_BPW, ROW), jnp.float32),
        pltpu.SemaphoreType.DMA,
        pltpu.SemaphoreType.DMA,
        pltpu.SemaphoreType.DMA,
        pltpu.SemaphoreType.DMA,
        pltpu.SemaphoreType.DMA,
    ],
)
def _sc_gather(tab_hbm, idxp_hbm, out_hbm, idx_v, rows_v, gsem0, gsem1,
               gsem2, gsem3, wsem):
    wid = lax.axis_index("s") * _NC + lax.axis_index("c")
    base = wid * _BPW
    q = _BPW // 4
    gsems = [gsem0, gsem1, gsem2, gsem3]
    pltpu.sync_copy(idxp_hbm.at[wid], idx_v)
    gh = [pltpu.async_copy(tab_hbm.at[idx_v.at[pl.ds(c * q, q)]],
                           rows_v.at[pl.ds(c * q, q)], gsems[c])
          for c in range(4)]
    handles = []
    for c in range(4):
        gh[c].wait()
        for s in range(3):
            handles.append(pltpu.async_copy(
                rows_v.at[pl.ds(c * q, q), pl.ds(s * SEC, SEC)],
                out_hbm.at[pl.ds(base + c * q, q), s, pl.ds(0, SEC)], wsem))
    for h in handles:
        h.wait()


VBLK = 512
NBLK = 8


def _assemble_body(sd_ref, g_ref, comp_ref, prep_ref):
    i = pl.program_id(0)
    sdh = sd_ref[:, :, 0:N_FIXED]
    gg = g_ref[:, :, OFF:OFF + N_SD]
    comp = jnp.concatenate([sdh, gg], axis=2)
    comp_ref[...] = comp
    flat = comp.reshape(VBLK * 3, 30)
    r = lax.broadcasted_iota(jnp.int32, (30, 30), 0)
    c = lax.broadcasted_iota(jnp.int32, (30, 30), 1)
    eye = (r == c).astype(jnp.float32)
    pblk = lax.dot_general(
        eye, flat, (((1,), (1,)), ((), ())),
        preferred_element_type=jnp.float32,
    )
    tail = N_VERTS * 3 - (NBLK - 1) * VBLK * 3

    @pl.when(i < NBLK - 1)
    def _():
        prep_ref[:, pl.ds(i * VBLK * 3, VBLK * 3)] = pblk

    @pl.when(i == NBLK - 1)
    def _():
        prep_ref[:, pl.ds((NBLK - 1) * VBLK * 3, tail)] = pblk[:, 0:tail]


def kernel(c0, c2, l0, l1, l2, sd, inds_back):
    params = jnp.concatenate([c0, c2, l0, l1, l2], axis=0)
    idx1d = inds_back.astype(jnp.int32)
    tab, idxp = pl.pallas_call(
        _build_table_body,
        out_shape=(
            jax.ShapeDtypeStruct((N_VERTS, ROW), jnp.float32),
            jax.ShapeDtypeStruct((_NW, _BPW), jnp.int32),
        ),
    )(params, idx1d)

    g = _sc_gather(tab, idxp)

    comp, prep = pl.pallas_call(
        _assemble_body,
        out_shape=(
            jax.ShapeDtypeStruct((N_VERTS, 3, 30), jnp.float32),
            jax.ShapeDtypeStruct((30, N_VERTS * 3), jnp.float32),
        ),
        grid=(NBLK,),
        in_specs=[
            pl.BlockSpec((VBLK, 3, 30), lambda i: (i, 0, 0)),
            pl.BlockSpec((VBLK, 3, ROW), lambda i: (i, 0, 0)),
        ],
        out_specs=(
            pl.BlockSpec((VBLK, 3, 30), lambda i: (i, 0, 0)),
            pl.BlockSpec((30, N_VERTS * 3), lambda i: (0, 0)),
        ),
    )(sd, g)
    return comp, prep


# speedup vs baseline: 1.0029x; 1.0029x over previous
import functools

import jax
import jax.numpy as jnp
from jax import lax
from jax.experimental import pallas as pl
from jax.experimental.pallas import tpu as pltpu
from jax.experimental.pallas import tpu_sc as plsc

N_VERTS = 3889
N_CENTER = 889
N_LEFT = 1500
N_SD = 20
N_FIXED = 10
SEC = 32
OFF = 10
ROW = 128
PAD_B = 4096

_info = plsc.get_sparse_core_info()
_NC = _info.num_cores
_NS = _info.num_subcores
_NW = _NC * _NS
_BPW = PAD_B // _NW


def _build_table_body(params_ref, idx_ref, tab_ref, idxp_ref):
    a, b = N_CENTER, N_CENTER + N_LEFT
    c0 = params_ref[0:889, :]
    c2 = params_ref[889:1778, :]
    l0 = params_ref[1778:3278, :]
    l1 = params_ref[3278:4778, :]
    l2 = params_ref[4778:6278, :]
    s0, s1, s2 = OFF, SEC + OFF, 2 * SEC + OFF
    tab_ref[0:a, s0:s0 + N_SD] = c0
    tab_ref[0:a, s1:s1 + N_SD] = jnp.zeros((N_CENTER, N_SD), jnp.float32)
    tab_ref[0:a, s2:s2 + N_SD] = c2
    tab_ref[a:b, s0:s0 + N_SD] = l0
    tab_ref[a:b, s1:s1 + N_SD] = l1
    tab_ref[a:b, s2:s2 + N_SD] = l2
    tab_ref[b:N_VERTS, s0:s0 + N_SD] = l0
    tab_ref[b:N_VERTS, s1:s1 + N_SD] = -l1
    tab_ref[b:N_VERTS, s2:s2 + N_SD] = l2
    ipad = jnp.concatenate(
        [idx_ref[...], jnp.zeros((PAD_B - N_VERTS,), jnp.int32)], axis=0)
    idxp_ref[...] = ipad.reshape(_NW, _BPW)


_sc_mesh = plsc.VectorSubcoreMesh(core_axis_name="c", subcore_axis_name="s")


@functools.partial(
    pl.kernel,
    mesh=_sc_mesh,
    out_type=jax.ShapeDtypeStruct((PAD_B, 3, ROW), jnp.float32),
    scratch_types=[
        pltpu.VMEM((_BPW,), jnp.int32),
        pltpu.VMEM((_BPW, ROW), jnp.float32),
        pltpu.SemaphoreType.DMA,
        pltpu.SemaphoreType.DMA,
        pltpu.SemaphoreType.DMA,
    ],
)
def _sc_gather(tab_hbm, idxp_hbm, out_hbm, idx_v, rows_v, gsem0, gsem1, wsem):
    wid = lax.axis_index("s") * _NC + lax.axis_index("c")
    base = wid * _BPW
    half = _BPW // 2
    pltpu.sync_copy(idxp_hbm.at[wid], idx_v)
    g0 = pltpu.async_copy(tab_hbm.at[idx_v.at[pl.ds(0, half)]],
                          rows_v.at[pl.ds(0, half)], gsem0)
    g1 = pltpu.async_copy(tab_hbm.at[idx_v.at[pl.ds(half, half)]],
                          rows_v.at[pl.ds(half, half)], gsem1)
    handles = []
    g0.wait()
    for s in range(3):
        handles.append(pltpu.async_copy(
            rows_v.at[pl.ds(0, half), pl.ds(s * SEC, SEC)],
            out_hbm.at[pl.ds(base, half), s, pl.ds(0, SEC)], wsem))
    g1.wait()
    for s in range(3):
        handles.append(pltpu.async_copy(
            rows_v.at[pl.ds(half, half), pl.ds(s * SEC, SEC)],
            out_hbm.at[pl.ds(base + half, half), s, pl.ds(0, SEC)], wsem))
    for h in handles:
        h.wait()


VBLK = 512
NBLK = 8


def _assemble_body(sd_ref, g_ref, comp_ref, prep_ref):
    i = pl.program_id(0)
    sdh = sd_ref[:, :, 0:N_FIXED]
    gg = g_ref[:, :, OFF:OFF + N_SD]
    comp = jnp.concatenate([sdh, gg], axis=2)
    comp_ref[...] = comp
    flat = comp.reshape(VBLK * 3, 30)
    r = lax.broadcasted_iota(jnp.int32, (30, 30), 0)
    c = lax.broadcasted_iota(jnp.int32, (30, 30), 1)
    eye = (r == c).astype(jnp.float32)
    pblk = lax.dot_general(
        eye, flat, (((1,), (1,)), ((), ())),
        preferred_element_type=jnp.float32,
    )
    tail = N_VERTS * 3 - (NBLK - 1) * VBLK * 3

    @pl.when(i < NBLK - 1)
    def _():
        prep_ref[:, pl.ds(i * VBLK * 3, VBLK * 3)] = pblk

    @pl.when(i == NBLK - 1)
    def _():
        prep_ref[:, pl.ds((NBLK - 1) * VBLK * 3, tail)] = pblk[:, 0:tail]


def kernel(c0, c2, l0, l1, l2, sd, inds_back):
    params = jnp.concatenate([c0, c2, l0, l1, l2], axis=0)
    idx1d = inds_back.astype(jnp.int32)
    tab, idxp = pl.pallas_call(
        _build_table_body,
        out_shape=(
            jax.ShapeDtypeStruct((N_VERTS, ROW), jnp.float32),
            jax.ShapeDtypeStruct((_NW, _BPW), jnp.int32),
        ),
    )(params, idx1d)

    g = _sc_gather(tab, idxp)

    comp, prep = pl.pallas_call(
        _assemble_body,
        out_shape=(
            jax.ShapeDtypeStruct((N_VERTS, 3, 30), jnp.float32),
            jax.ShapeDtypeStruct((30, N_VERTS * 3), jnp.float32),
        ),
        grid=(NBLK,),
        in_specs=[
            pl.BlockSpec((VBLK, 3, 30), lambda i: (i, 0, 0)),
            pl.BlockSpec((VBLK, 3, ROW), lambda i: (i, 0, 0)),
        ],
        out_specs=(
            pl.BlockSpec((VBLK, 3, 30), lambda i: (i, 0, 0)),
            pl.BlockSpec((30, N_VERTS * 3), lambda i: (0, 0)),
        ),
    )(sd, g)
    return comp, prep
